# baseline (speedup 1.0000x reference)
import functools

import jax
import jax.numpy as jnp
from jax import lax
from jax.experimental import pallas as pl
from jax.experimental.pallas import tpu as pltpu
from jax.experimental.pallas import tpu_sc as plsc

N = 10000
E = 320000
D = 128

NC = 2
NS = 16
NW = NC * NS
EPT = E // NW
K = 40
NCHUNK = EPT // K
NBUF = 8
G = 4
W = NBUF - G - 1
RPT = 624
TAIL = N - NS * RPT

_ROW_BLK = 1000


def _expmap_body(x_ref, o_ref):
    x = x_ref[...]
    n = jnp.sqrt(jnp.sum(x * x, axis=-1, keepdims=True))
    o_ref[...] = jnp.tanh(n) * x / (n + 1e-8)


_expmap_call = pl.pallas_call(
    _expmap_body,
    grid=(N // _ROW_BLK,),
    in_specs=[pl.BlockSpec((_ROW_BLK, D), lambda i: (i, 0))],
    out_specs=pl.BlockSpec((_ROW_BLK, D), lambda i: (i, 0)),
    out_shape=jax.ShapeDtypeStruct((N, D), jnp.float32),
)


def _logmap_body(p0_ref, p1_ref, b_ref, o_ref):
    y = p0_ref[...] + p1_ref[...]
    n = jnp.sqrt(jnp.sum(y * y, axis=-1, keepdims=True))
    atanh_n = 0.5 * jnp.log((1.0 + n) / (1.0 - n))
    o_ref[...] = atanh_n * y / (n + 1e-8) + b_ref[...]


_logmap_call = pl.pallas_call(
    _logmap_body,
    grid=(N // _ROW_BLK,),
    in_specs=[
        pl.BlockSpec((_ROW_BLK, D), lambda i: (i, 0)),
        pl.BlockSpec((_ROW_BLK, D), lambda i: (i, 0)),
        pl.BlockSpec((1, D), lambda i: (0, 0)),
    ],
    out_specs=pl.BlockSpec((_ROW_BLK, D), lambda i: (i, 0)),
    out_shape=jax.ShapeDtypeStruct((N, D), jnp.float32),
)


_mesh = plsc.VectorSubcoreMesh(core_axis_name="c", subcore_axis_name="s")


@functools.partial(
    pl.kernel,
    mesh=_mesh,
    out_type=jax.ShapeDtypeStruct((NC * N, D), jnp.float32),
    scratch_types=(
        [pltpu.VMEM_SHARED((N, D), jnp.float32)]
        + [pltpu.VMEM((K, D), jnp.float32) for _ in range(NBUF)]
        + [pltpu.VMEM((1, K), jnp.int32) for _ in range(NBUF)]
        + [pltpu.VMEM((1, K), jnp.int32) for _ in range(NBUF)]
        + [pltpu.VMEM((1, K), jnp.float32) for _ in range(NBUF)]
        + [pltpu.SemaphoreType.DMA for _ in range(3 * NBUF)]
    ),
)
def _sc_segsum(xp_hbm, src_hbm, dst_hbm, w_hbm, zero_hbm, out_hbm,
               accum, *bufs_and_sems):
    rows = bufs_and_sems[0 * NBUF:1 * NBUF]
    srcb = bufs_and_sems[1 * NBUF:2 * NBUF]
    dstb = bufs_and_sems[2 * NBUF:3 * NBUF]
    wb = bufs_and_sems[3 * NBUF:4 * NBUF]
    sem_g = bufs_and_sems[4 * NBUF:5 * NBUF]
    sem_s = bufs_and_sems[5 * NBUF:6 * NBUF]
    sem_m = bufs_and_sems[6 * NBUF:7 * NBUF]

    c = lax.axis_index("c")
    s = lax.axis_index("s")
    wid = c * NS + s

    def meta_start(ci, b):
        pltpu.async_copy(src_hbm.at[wid, pl.ds(ci, 1)], srcb[b], sem_m[b])
        pltpu.async_copy(dst_hbm.at[wid, pl.ds(ci, 1)], dstb[b], sem_m[b])
        pltpu.async_copy(w_hbm.at[wid, pl.ds(ci, 1)], wb[b], sem_m[b])

    def meta_wait(ci, b):
        pltpu.make_async_copy(src_hbm.at[wid, pl.ds(ci, 1)], srcb[b],
                              sem_m[b]).wait()
        pltpu.make_async_copy(dst_hbm.at[wid, pl.ds(ci, 1)], dstb[b],
                              sem_m[b]).wait()
        pltpu.make_async_copy(w_hbm.at[wid, pl.ds(ci, 1)], wb[b],
                              sem_m[b]).wait()

    def gather_start(b):
        pltpu.async_copy(xp_hbm.at[srcb[b].at[0]], rows[b], sem_g[b])

    def scatter_wait(b):
        pltpu.make_async_copy(rows[b], accum.at[dstb[b].at[0]],
                              sem_s[b]).wait()

    for j in range(G + 1):
        meta_start(j, j)

    pltpu.sync_copy(zero_hbm.at[pl.ds(0, RPT)], accum.at[pl.ds(s * RPT, RPT)])

    @pl.when(s == 0)
    def _zero_tail():
        pltpu.sync_copy(zero_hbm.at[pl.ds(0, TAIL)],
                        accum.at[pl.ds(NS * RPT, TAIL)])

    plsc.subcore_barrier()

    for j in range(G):
        meta_wait(j, j)
        gather_start(j)

    def do_chunk(ci, b):
        bg = (b + G) % NBUF
        bp = (b + G + 1) % NBUF

        @pl.when(ci >= W)
        def _wait_scatter():
            scatter_wait((b - W) % NBUF)

        @pl.when(ci + G < NCHUNK)
        def _next_gather():
            meta_wait(ci + G, bg)
            gather_start(bg)

        @pl.when(ci + G + 1 < NCHUNK)
        def _next_meta():
            meta_start(ci + G + 1, bp)

        pltpu.make_async_copy(xp_hbm.at[srcb[b].at[0]], rows[b],
                              sem_g[b]).wait()

        def group(g, carry):
            w16 = wb[b][0, pl.ds(g * 16, 16)]
            for e in range(16):
                wspl = jnp.full((16,), w16[e])
                for v in range(D // 16):
                    sl = pl.ds(v * 16, 16)
                    rows[b][g * 16 + e, sl] = rows[b][g * 16 + e, sl] * wspl
            return carry

        lax.fori_loop(0, K // 16, group, 0)
        if K % 16 != 0:
            base = K - (K % 16)
            woff = K - 16
            w16 = wb[b][0, pl.ds(woff, 16)]
            for e in range(base - woff, 16):
                wspl = jnp.full((16,), w16[e])
                for v in range(D // 16):
                    sl = pl.ds(v * 16, 16)
                    rows[b][woff + e, sl] = rows[b][woff + e, sl] * wspl

        pltpu.async_copy(rows[b], accum.at[dstb[b].at[0]], sem_s[b],
                         add=True)

    def super_chunk(si, carry):
        for b in range(NBUF):
            ci = si * NBUF + b

            @pl.when(ci < NCHUNK)
            def _body():
                do_chunk(ci, b)

            del _body
        return carry

    lax.fori_loop(0, (NCHUNK + NBUF - 1) // NBUF, super_chunk, 0)

    for ci in range(NCHUNK - W, NCHUNK):
        scatter_wait(ci % NBUF)
    plsc.subcore_barrier()

    pltpu.sync_copy(accum.at[pl.ds(s * RPT, RPT)],
                    out_hbm.at[pl.ds(c * N + s * RPT, RPT)])

    @pl.when(s == 0)
    def _out_tail():
        pltpu.sync_copy(accum.at[pl.ds(NS * RPT, TAIL)],
                        out_hbm.at[pl.ds(c * N + NS * RPT, TAIL)])


def kernel(x, edge_index, edge_weight, bias):
    x_proj = _expmap_call(x)
    src = edge_index[1].reshape(NW, NCHUNK, K)
    dst = edge_index[0].reshape(NW, NCHUNK, K)
    w = edge_weight.reshape(NW, NCHUNK, K)
    zeros = jnp.zeros((RPT, D), jnp.float32)
    partial = _sc_segsum(x_proj, src, dst, w, zeros)
    return _logmap_call(partial[:N], partial[N:], bias.reshape(1, D))


# speedup vs baseline: 1.2356x; 1.2356x over previous
import functools

import jax
import jax.numpy as jnp
from jax import lax
from jax.experimental import pallas as pl
from jax.experimental.pallas import tpu as pltpu
from jax.experimental.pallas import tpu_sc as plsc

N = 10000
E = 320000
D = 128

NC = 2
NS = 16
NW = NC * NS
EPT = E // NW
K = 80
NCHUNK = EPT // K
NBUF = 4
G = 2
W = NBUF - G - 1
RPT = 624
TAIL = N - NS * RPT

_ROW_BLK = 1000


def _expmap_body(x_ref, o_ref):
    x = x_ref[...]
    n = jnp.sqrt(jnp.sum(x * x, axis=-1, keepdims=True))
    o_ref[...] = jnp.tanh(n) * x / (n + 1e-8)


_expmap_call = pl.pallas_call(
    _expmap_body,
    grid=(N // _ROW_BLK,),
    in_specs=[pl.BlockSpec((_ROW_BLK, D), lambda i: (i, 0))],
    out_specs=pl.BlockSpec((_ROW_BLK, D), lambda i: (i, 0)),
    out_shape=jax.ShapeDtypeStruct((N, D), jnp.float32),
)


def _logmap_body(p0_ref, p1_ref, b_ref, o_ref):
    y = p0_ref[...] + p1_ref[...]
    n = jnp.sqrt(jnp.sum(y * y, axis=-1, keepdims=True))
    atanh_n = 0.5 * jnp.log((1.0 + n) / (1.0 - n))
    o_ref[...] = atanh_n * y / (n + 1e-8) + b_ref[...]


_logmap_call = pl.pallas_call(
    _logmap_body,
    grid=(N // _ROW_BLK,),
    in_specs=[
        pl.BlockSpec((_ROW_BLK, D), lambda i: (i, 0)),
        pl.BlockSpec((_ROW_BLK, D), lambda i: (i, 0)),
        pl.BlockSpec((1, D), lambda i: (0, 0)),
    ],
    out_specs=pl.BlockSpec((_ROW_BLK, D), lambda i: (i, 0)),
    out_shape=jax.ShapeDtypeStruct((N, D), jnp.float32),
)


_mesh = plsc.VectorSubcoreMesh(core_axis_name="c", subcore_axis_name="s")


@functools.partial(
    pl.kernel,
    mesh=_mesh,
    out_type=jax.ShapeDtypeStruct((NC * N, D), jnp.float32),
    scratch_types=(
        [pltpu.VMEM_SHARED((N, D), jnp.float32)]
        + [pltpu.VMEM((K, D), jnp.float32) for _ in range(NBUF)]
        + [pltpu.VMEM((1, K), jnp.int32) for _ in range(NBUF)]
        + [pltpu.VMEM((1, K), jnp.int32) for _ in range(NBUF)]
        + [pltpu.VMEM((1, K), jnp.float32) for _ in range(NBUF)]
        + [pltpu.SemaphoreType.DMA for _ in range(3 * NBUF)]
    ),
)
def _sc_segsum(xp_hbm, src_hbm, dst_hbm, w_hbm, zero_hbm, out_hbm,
               accum, *bufs_and_sems):
    rows = bufs_and_sems[0 * NBUF:1 * NBUF]
    srcb = bufs_and_sems[1 * NBUF:2 * NBUF]
    dstb = bufs_and_sems[2 * NBUF:3 * NBUF]
    wb = bufs_and_sems[3 * NBUF:4 * NBUF]
    sem_g = bufs_and_sems[4 * NBUF:5 * NBUF]
    sem_s = bufs_and_sems[5 * NBUF:6 * NBUF]
    sem_m = bufs_and_sems[6 * NBUF:7 * NBUF]

    c = lax.axis_index("c")
    s = lax.axis_index("s")
    wid = c * NS + s

    def meta_start(ci, b):
        pltpu.async_copy(src_hbm.at[wid, pl.ds(ci, 1)], srcb[b], sem_m[b])
        pltpu.async_copy(dst_hbm.at[wid, pl.ds(ci, 1)], dstb[b], sem_m[b])
        pltpu.async_copy(w_hbm.at[wid, pl.ds(ci, 1)], wb[b], sem_m[b])

    def meta_wait(ci, b):
        pltpu.make_async_copy(src_hbm.at[wid, pl.ds(ci, 1)], srcb[b],
                              sem_m[b]).wait()
        pltpu.make_async_copy(dst_hbm.at[wid, pl.ds(ci, 1)], dstb[b],
                              sem_m[b]).wait()
        pltpu.make_async_copy(w_hbm.at[wid, pl.ds(ci, 1)], wb[b],
                              sem_m[b]).wait()

    def gather_start(b):
        pltpu.async_copy(xp_hbm.at[srcb[b].at[0]], rows[b], sem_g[b])

    def scatter_wait(b):
        pltpu.make_async_copy(rows[b], accum.at[dstb[b].at[0]],
                              sem_s[b]).wait()

    for j in range(G + 1):
        meta_start(j, j)

    pltpu.sync_copy(zero_hbm.at[pl.ds(0, RPT)], accum.at[pl.ds(s * RPT, RPT)])

    @pl.when(s == 0)
    def _zero_tail():
        pltpu.sync_copy(zero_hbm.at[pl.ds(0, TAIL)],
                        accum.at[pl.ds(NS * RPT, TAIL)])

    plsc.subcore_barrier()

    for j in range(G):
        meta_wait(j, j)
        gather_start(j)

    def do_chunk(ci, b):
        bg = (b + G) % NBUF
        bp = (b + G + 1) % NBUF

        @pl.when(ci >= W)
        def _wait_scatter():
            scatter_wait((b - W) % NBUF)

        @pl.when(ci + G < NCHUNK)
        def _next_gather():
            meta_wait(ci + G, bg)
            gather_start(bg)

        @pl.when(ci + G + 1 < NCHUNK)
        def _next_meta():
            meta_start(ci + G + 1, bp)

        pltpu.make_async_copy(xp_hbm.at[srcb[b].at[0]], rows[b],
                              sem_g[b]).wait()

        def group(g, carry):
            w16 = wb[b][0, pl.ds(g * 16, 16)]
            for e in range(16):
                wspl = jnp.full((16,), w16[e])
                for v in range(D // 16):
                    sl = pl.ds(v * 16, 16)
                    rows[b][g * 16 + e, sl] = rows[b][g * 16 + e, sl] * wspl
            return carry

        lax.fori_loop(0, K // 16, group, 0)
        if K % 16 != 0:
            base = K - (K % 16)
            woff = K - 16
            w16 = wb[b][0, pl.ds(woff, 16)]
            for e in range(base - woff, 16):
                wspl = jnp.full((16,), w16[e])
                for v in range(D // 16):
                    sl = pl.ds(v * 16, 16)
                    rows[b][woff + e, sl] = rows[b][woff + e, sl] * wspl

        pltpu.async_copy(rows[b], accum.at[dstb[b].at[0]], sem_s[b],
                         add=True)

    def super_chunk(si, carry):
        for b in range(NBUF):
            ci = si * NBUF + b

            @pl.when(ci < NCHUNK)
            def _body():
                do_chunk(ci, b)

            del _body
        return carry

    lax.fori_loop(0, (NCHUNK + NBUF - 1) // NBUF, super_chunk, 0)

    for ci in range(NCHUNK - W, NCHUNK):
        scatter_wait(ci % NBUF)
    plsc.subcore_barrier()

    pltpu.sync_copy(accum.at[pl.ds(s * RPT, RPT)],
                    out_hbm.at[pl.ds(c * N + s * RPT, RPT)])

    @pl.when(s == 0)
    def _out_tail():
        pltpu.sync_copy(accum.at[pl.ds(NS * RPT, TAIL)],
                        out_hbm.at[pl.ds(c * N + NS * RPT, TAIL)])


def kernel(x, edge_index, edge_weight, bias):
    x_proj = _expmap_call(x)
    src = edge_index[1].reshape(NW, NCHUNK, K)
    dst = edge_index[0].reshape(NW, NCHUNK, K)
    w = edge_weight.reshape(NW, NCHUNK, K)
    zeros = jnp.zeros((RPT, D), jnp.float32)
    partial = _sc_segsum(x_proj, src, dst, w, zeros)
    return _logmap_call(partial[:N], partial[N:], bias.reshape(1, D))


# speedup vs baseline: 1.3443x; 1.0880x over previous
import functools

import jax
import jax.numpy as jnp
from jax import lax
from jax.experimental import pallas as pl
from jax.experimental.pallas import tpu as pltpu
from jax.experimental.pallas import tpu_sc as plsc

N = 10000
E = 320000
D = 128

NC = 2
NS = 16
NW = NC * NS
EPT = E // NW
K = 80
NCHUNK = EPT // K
NBUF = 4
G = 2
W = NBUF - G - 1
RPT = 624
TAIL = N - NS * RPT

_ROW_BLK = 2000


def _expmap_body(x_ref, o_ref):
    x = x_ref[...]
    n = jnp.sqrt(jnp.sum(x * x, axis=-1, keepdims=True))
    o_ref[...] = jnp.tanh(n) * x / (n + 1e-8)


_expmap_call = pl.pallas_call(
    _expmap_body,
    grid=(N // _ROW_BLK,),
    in_specs=[pl.BlockSpec((_ROW_BLK, D), lambda i: (i, 0))],
    out_specs=pl.BlockSpec((_ROW_BLK, D), lambda i: (i, 0)),
    out_shape=jax.ShapeDtypeStruct((N, D), jnp.float32),
)


def _logmap_body(p0_ref, p1_ref, b_ref, o_ref):
    y = p0_ref[...] + p1_ref[...]
    n = jnp.sqrt(jnp.sum(y * y, axis=-1, keepdims=True))
    atanh_n = 0.5 * jnp.log((1.0 + n) / (1.0 - n))
    o_ref[...] = atanh_n * y / (n + 1e-8) + b_ref[...]


_logmap_call = pl.pallas_call(
    _logmap_body,
    grid=(N // _ROW_BLK,),
    in_specs=[
        pl.BlockSpec((_ROW_BLK, D), lambda i: (i, 0)),
        pl.BlockSpec((_ROW_BLK, D), lambda i: (i + N // _ROW_BLK, 0)),
        pl.BlockSpec((1, D), lambda i: (0, 0)),
    ],
    out_specs=pl.BlockSpec((_ROW_BLK, D), lambda i: (i, 0)),
    out_shape=jax.ShapeDtypeStruct((N, D), jnp.float32),
)


_mesh = plsc.VectorSubcoreMesh(core_axis_name="c", subcore_axis_name="s")


@functools.partial(
    pl.kernel,
    mesh=_mesh,
    out_type=jax.ShapeDtypeStruct((NC * N, D), jnp.float32),
    scratch_types=(
        [pltpu.VMEM_SHARED((N, D), jnp.float32)]
        + [pltpu.VMEM((K, D), jnp.float32) for _ in range(NBUF)]
        + [pltpu.VMEM((K,), jnp.int32) for _ in range(NBUF)]
        + [pltpu.VMEM((K,), jnp.int32) for _ in range(NBUF)]
        + [pltpu.VMEM((K,), jnp.float32) for _ in range(NBUF)]
        + [pltpu.SemaphoreType.DMA for _ in range(3 * NBUF)]
    ),
)
def _sc_segsum(xp_hbm, src_hbm, dst_hbm, w_hbm, zero_hbm, out_hbm,
               accum, *bufs_and_sems):
    rows = bufs_and_sems[0 * NBUF:1 * NBUF]
    srcb = bufs_and_sems[1 * NBUF:2 * NBUF]
    dstb = bufs_and_sems[2 * NBUF:3 * NBUF]
    wb = bufs_and_sems[3 * NBUF:4 * NBUF]
    sem_g = bufs_and_sems[4 * NBUF:5 * NBUF]
    sem_s = bufs_and_sems[5 * NBUF:6 * NBUF]
    sem_m = bufs_and_sems[6 * NBUF:7 * NBUF]

    c = lax.axis_index("c")
    s = lax.axis_index("s")
    wid = c * NS + s

    def meta_start(ci, b):
        off = pl.multiple_of(wid * EPT + ci * K, 8)
        pltpu.async_copy(src_hbm.at[pl.ds(off, K)], srcb[b], sem_m[b])
        pltpu.async_copy(dst_hbm.at[pl.ds(off, K)], dstb[b], sem_m[b])
        pltpu.async_copy(w_hbm.at[pl.ds(off, K)], wb[b], sem_m[b])

    def meta_wait(ci, b):
        off = pl.multiple_of(wid * EPT + ci * K, 8)
        pltpu.make_async_copy(src_hbm.at[pl.ds(off, K)], srcb[b],
                              sem_m[b]).wait()
        pltpu.make_async_copy(dst_hbm.at[pl.ds(off, K)], dstb[b],
                              sem_m[b]).wait()
        pltpu.make_async_copy(w_hbm.at[pl.ds(off, K)], wb[b],
                              sem_m[b]).wait()

    def gather_start(b):
        pltpu.async_copy(xp_hbm.at[srcb[b]], rows[b], sem_g[b])

    def scatter_wait(b):
        pltpu.make_async_copy(rows[b], accum.at[dstb[b]],
                              sem_s[b]).wait()

    for j in range(G + 1):
        meta_start(j, j)

    pltpu.sync_copy(zero_hbm.at[pl.ds(0, RPT)], accum.at[pl.ds(s * RPT, RPT)])

    @pl.when(s == 0)
    def _zero_tail():
        pltpu.sync_copy(zero_hbm.at[pl.ds(0, TAIL)],
                        accum.at[pl.ds(NS * RPT, TAIL)])

    plsc.subcore_barrier()

    for j in range(G):
        meta_wait(j, j)
        gather_start(j)

    def do_chunk(ci, b):
        bg = (b + G) % NBUF
        bp = (b + G + 1) % NBUF

        @pl.when(ci >= W)
        def _wait_scatter():
            scatter_wait((b - W) % NBUF)

        @pl.when(ci + G < NCHUNK)
        def _next_gather():
            meta_wait(ci + G, bg)
            gather_start(bg)

        @pl.when(ci + G + 1 < NCHUNK)
        def _next_meta():
            meta_start(ci + G + 1, bp)

        pltpu.make_async_copy(xp_hbm.at[srcb[b]], rows[b],
                              sem_g[b]).wait()

        def group(g, carry):
            w16 = wb[b][pl.ds(g * 16, 16)]
            for e in range(16):
                wspl = jnp.full((16,), w16[e])
                for v in range(D // 16):
                    sl = pl.ds(v * 16, 16)
                    rows[b][g * 16 + e, sl] = rows[b][g * 16 + e, sl] * wspl
            return carry

        lax.fori_loop(0, K // 16, group, 0)
        if K % 16 != 0:
            base = K - (K % 16)
            woff = K - 16
            w16 = wb[b][pl.ds(woff, 16)]
            for e in range(base - woff, 16):
                wspl = jnp.full((16,), w16[e])
                for v in range(D // 16):
                    sl = pl.ds(v * 16, 16)
                    rows[b][woff + e, sl] = rows[b][woff + e, sl] * wspl

        pltpu.async_copy(rows[b], accum.at[dstb[b]], sem_s[b],
                         add=True)

    def super_chunk(si, carry):
        for b in range(NBUF):
            ci = si * NBUF + b

            @pl.when(ci < NCHUNK)
            def _body():
                do_chunk(ci, b)

            del _body
        return carry

    lax.fori_loop(0, (NCHUNK + NBUF - 1) // NBUF, super_chunk, 0)

    for ci in range(NCHUNK - W, NCHUNK):
        scatter_wait(ci % NBUF)
    plsc.subcore_barrier()

    pltpu.sync_copy(accum.at[pl.ds(s * RPT, RPT)],
                    out_hbm.at[pl.ds(c * N + s * RPT, RPT)])

    @pl.when(s == 0)
    def _out_tail():
        pltpu.sync_copy(accum.at[pl.ds(NS * RPT, TAIL)],
                        out_hbm.at[pl.ds(c * N + NS * RPT, TAIL)])


def kernel(x, edge_index, edge_weight, bias):
    x_proj = _expmap_call(x)
    src = edge_index[1]
    dst = edge_index[0]
    w = edge_weight
    zeros = jnp.zeros((RPT, D), jnp.float32)
    partial = _sc_segsum(x_proj, src, dst, w, zeros)
    return _logmap_call(partial, partial, bias.reshape(1, D))
